# baseline (speedup 1.0000x reference)
import jax
import jax.numpy as jnp
from jax.experimental import pallas as pl
from functools import partial

N = 10000


def _gram_body(h_ref, ht_ref, out_ref):
    out_ref[...] = jnp.dot(h_ref[...], ht_ref[...],
                           preferred_element_type=jnp.float32)


def _gram(h):
    ht = h.T
    return pl.pallas_call(
        _gram_body,
        grid=(10, 10),
        in_specs=[
            pl.BlockSpec((1024, 128), lambda i, j: (i, 0)),
            pl.BlockSpec((128, 1024), lambda i, j: (0, j)),
        ],
        out_specs=pl.BlockSpec((1024, 1024), lambda i, j: (i, j)),
        out_shape=jax.ShapeDtypeStruct((N, N), jnp.float32),
    )(h, ht)


def kernel(x, edge_index, edge_attr, edge_type, emb, W0, q0, k0, We0, e0, b0,
           W1, q1, k1, We1, e1, b1, W2, q2, k2, We2, e2, b2,
           W3, q3, k3, We3, e3, b3, W4, q4, k4, We4, e4, b4):
    src, dst = edge_index[0], edge_index[1]
    h = emb[x[:, 0]]
    params = [(W0, q0, k0, We0, e0, b0), (W1, q1, k1, We1, e1, b1),
              (W2, q2, k2, We2, e2, b2), (W3, q3, k3, We3, e3, b3),
              (W4, q4, k4, We4, e4, b4)]
    for i, (W, q, k, We, e, b) in enumerate(params):
        hw = h @ W
        aq = (hw @ q)[:, 0]
        ak = (hw @ k)[:, 0]
        ae = (edge_attr @ (We.T @ e))[:, 0]
        shift_ub = jnp.max(aq) + jnp.max(ak) + jnp.max(ae)
        shift = jnp.maximum(shift_ub, 0.2 * shift_ub)
        alpha = aq[dst] + ak[src] + ae
        alpha = jnp.where(alpha >= 0, alpha, 0.2 * alpha)
        ex = jnp.exp(alpha - shift)
        num = jax.ops.segment_sum(ex[:, None] * hw[src], dst, num_segments=N)
        den = jax.ops.segment_sum(ex, dst, num_segments=N)
        h = jnp.where(den[:, None] > 0, num / den[:, None], 0.0) + b
        if i < 4:
            h = jnp.where(h >= 0, h, 0.01 * h)
    return _gram(h)


# speedup vs baseline: 5.6989x; 5.6989x over previous
import functools

import jax
import jax.numpy as jnp
from jax import lax
from jax.experimental import pallas as pl
from jax.experimental.pallas import tpu as pltpu
from jax.experimental.pallas import tpu_sc as plsc

N = 10000
E = 320000
NC, NS = 2, 16
NW = NC * NS
EP = 327680
EW = EP // NW
K = 64
NCHUNK = EW // K
NPAD = 10240
NEG = -1.0e30
CW = 128

_MESH = plsc.VectorSubcoreMesh(core_axis_name="c", subcore_axis_name="s",
                               num_cores=NC, num_subcores=NS)
_SC_PARAMS = pltpu.CompilerParams(needs_layout_passes=False)


def _ae_body(ea_ref, wt0, e0, wt1, e1, wt2, e2, wt3, e3, wt4, e4,
             ae_ref, am_ref):
    cols = []
    for wt, ev in ((wt0, e0), (wt1, e1), (wt2, e2), (wt3, e3), (wt4, e4)):
        cols.append(jnp.dot(wt[...], ev[...],
                            preferred_element_type=jnp.float32))
    ce = jnp.concatenate(cols + [cols[0], cols[0], cols[0]], axis=1)
    blk = jnp.dot(ea_ref[...], ce, preferred_element_type=jnp.float32)
    ae_ref[...] = blk

    @pl.when(pl.program_id(0) == 0)
    def _():
        am_ref[...] = jnp.full((8, 128), NEG, jnp.float32)

    for l in range(5):
        ml = jnp.max(blk[:, l])
        prev = am_ref[pl.ds(l, 1), :]
        am_ref[pl.ds(l, 1), :] = jnp.maximum(prev, ml)


def _stage0(edge_attr, wts, es):
    EB = 20000
    args = []
    specs = [pl.BlockSpec((EB, 16), lambda i: (i, 0))]
    for wt, ev in zip(wts, es):
        args += [wt, ev]
        c = wt.shape[1]
        specs += [pl.BlockSpec((16, c), lambda i: (0, 0)),
                  pl.BlockSpec((c, 1), lambda i: (0, 0))]
    return pl.pallas_call(
        _ae_body,
        grid=(E // EB,),
        in_specs=specs,
        out_specs=[pl.BlockSpec((EB, 8), lambda i: (i, 0)),
                   pl.BlockSpec((8, 128), lambda i: (0, 0))],
        out_shape=[jax.ShapeDtypeStruct((E, 8), jnp.float32),
                   jax.ShapeDtypeStruct((8, 128), jnp.float32)],
    )(edge_attr, *args)


def _emb_body(tab_hbm, idx_hbm, out_hbm, idx_v, rows_v, sem):
    wid = lax.axis_index("c") * NS + lax.axis_index("s")

    def step(j, carry):
        base = wid * 320 + j * 80
        pltpu.sync_copy(idx_hbm.at[pl.ds(base, 80)], idx_v)
        pltpu.async_copy(tab_hbm.at[idx_v], rows_v, sem).wait()
        pltpu.sync_copy(rows_v, out_hbm.at[pl.ds(base, 80)])
        return carry

    lax.fori_loop(0, 4, step, 0)


def _emb_lookup(emb, idx_pad):
    f = pl.kernel(
        _emb_body,
        out_type=jax.ShapeDtypeStruct((NPAD, 128), jnp.float32),
        mesh=_MESH,
        compiler_params=_SC_PARAMS,
        scratch_types=[pltpu.VMEM((80,), jnp.int32),
                       pltpu.VMEM((80, 128), jnp.float32),
                       pltpu.SemaphoreType.DMA],
    )
    return f(emb, idx_pad)


def _prep_body(cout, h_ref, w_ref, q_ref, k_ref, am_ref,
               hwp_ref, aq_ref, ak_ref, sh_ref):
    hw = jnp.dot(h_ref[...], w_ref[...], preferred_element_type=jnp.float32)
    hwp_ref[:, :cout] = hw
    if cout < CW:
        hwp_ref[:, cout:] = jnp.zeros((N, CW - cout), jnp.float32)
    aq = jnp.dot(hw, q_ref[...], preferred_element_type=jnp.float32)
    ak = jnp.dot(hw, k_ref[...], preferred_element_type=jnp.float32)
    aq_ref[...] = aq
    ak_ref[...] = ak
    s = jnp.max(aq) + jnp.max(ak) + jnp.max(am_ref[pl.ds(0, 1), :])
    sh_ref[...] = jnp.full((8, 128), s, jnp.float32)


def _prep(h, W, q, k, am_row):
    cin, cout = W.shape
    cp = CW
    body = functools.partial(_prep_body, cout)
    return pl.pallas_call(
        body,
        in_specs=[pl.BlockSpec((N, cin), lambda: (0, 0)),
                  pl.BlockSpec((cin, cout), lambda: (0, 0)),
                  pl.BlockSpec((cout, 1), lambda: (0, 0)),
                  pl.BlockSpec((cout, 1), lambda: (0, 0)),
                  pl.BlockSpec((1, 128), lambda: (0, 0))],
        out_specs=[pl.BlockSpec((N, cp), lambda: (0, 0)),
                   pl.BlockSpec((N, 1), lambda: (0, 0)),
                   pl.BlockSpec((N, 1), lambda: (0, 0)),
                   pl.BlockSpec((8, 128), lambda: (0, 0))],
        out_shape=[jax.ShapeDtypeStruct((N, cp), jnp.float32),
                   jax.ShapeDtypeStruct((N, 1), jnp.float32),
                   jax.ShapeDtypeStruct((N, 1), jnp.float32),
                   jax.ShapeDtypeStruct((8, 128), jnp.float32)],
    )(h, W, q, k, am_row)


def _edge_body(cout, src_hbm, dst_hbm, ae_hbm, aq_hbm, ak_hbm, sh_hbm,
               hwp_hbm, accp_hbm, denp_hbm,
               acc_sh, den_sh, aq_v, ak_v, sh_v, src_v, dst_v, ae_v, ex_v,
               rows_v, zb_v, dzb_v, sem):
    cid = lax.axis_index("c")
    sid = lax.axis_index("s")
    wid = cid * NS + sid

    zero16 = jnp.zeros((16,), jnp.float32)
    for r in range(16):
        for c in range(CW // 16):
            zb_v[r, pl.ds(c * 16, 16)] = zero16

    def zclr(j, carry):
        dzb_v[pl.ds(j * 16, 16)] = zero16
        return carry

    lax.fori_loop(0, 40, zclr, 0)
    pltpu.sync_copy(dzb_v, den_sh.at[pl.ds(sid * 640, 640)])

    def zstep(j, carry):
        pltpu.sync_copy(zb_v, acc_sh.at[pl.ds(sid * 640 + j * 16, 16)])
        return carry

    lax.fori_loop(0, 40, zstep, 0)

    pltpu.sync_copy(aq_hbm, aq_v)
    pltpu.sync_copy(ak_hbm, ak_v)
    pltpu.sync_copy(sh_hbm.at[pl.ds(0, 16)], sh_v)
    shift = sh_v[...]
    plsc.subcore_barrier()

    ebase = wid * EW

    def chunk(t, carry):
        off = ebase + t * K
        pltpu.sync_copy(src_hbm.at[pl.ds(off, K)], src_v)
        pltpu.sync_copy(dst_hbm.at[pl.ds(off, K)], dst_v)
        pltpu.sync_copy(ae_hbm.at[pl.ds(off, K)], ae_v)
        gat = pltpu.async_copy(hwp_hbm.at[src_v], rows_v, sem)
        for g in range(K // 16):
            d16 = dst_v[pl.ds(g * 16, 16)]
            s16 = src_v[pl.ds(g * 16, 16)]
            a = (plsc.load_gather(aq_v, [d16])
                 + plsc.load_gather(ak_v, [s16])
                 + ae_v[pl.ds(g * 16, 16)])
            a = jnp.where(a >= 0.0, a, 0.2 * a)
            ex_v[pl.ds(g * 16, 16)] = jnp.exp(a - shift)
        gat.wait()

        def rgroup(gg, carry2):
            r0 = gg * 8
            for rr in range(8):
                r = r0 + rr
                w = plsc.load_gather(ex_v, [jnp.zeros((16,), jnp.int32) + r])
                for c in range(cout // 16):
                    sl = pl.ds(c * 16, 16)
                    rows_v[r, sl] = rows_v[r, sl] * w
            return carry2

        lax.fori_loop(0, K // 8, rgroup, 0)
        pltpu.sync_copy(rows_v, acc_sh.at[dst_v], add=True)
        pltpu.sync_copy(ex_v, den_sh.at[dst_v], add=True)
        return carry

    lax.fori_loop(0, NCHUNK, chunk, 0)
    plsc.subcore_barrier()
    pltpu.sync_copy(acc_sh.at[pl.ds(sid * 640, 640)],
                    accp_hbm.at[cid, pl.ds(sid * 640, 640)])
    pltpu.sync_copy(den_sh.at[pl.ds(sid * 640, 640)],
                    denp_hbm.at[cid, pl.ds(sid * 640, 640)])


def _edge(srcp, dstp, aep, aq, ak, sh, hwp, cout):
    body = functools.partial(_edge_body, cout)
    f = pl.kernel(
        body,
        out_type=[jax.ShapeDtypeStruct((NC, NPAD, CW), jnp.float32),
                  jax.ShapeDtypeStruct((NC, NPAD), jnp.float32)],
        mesh=_MESH,
        compiler_params=_SC_PARAMS,
        scratch_types=[pltpu.VMEM_SHARED((NPAD, CW), jnp.float32),
                       pltpu.VMEM_SHARED((NPAD,), jnp.float32),
                       pltpu.VMEM((N,), jnp.float32),
                       pltpu.VMEM((N,), jnp.float32),
                       pltpu.VMEM((16,), jnp.float32),
                       pltpu.VMEM((K,), jnp.int32),
                       pltpu.VMEM((K,), jnp.int32),
                       pltpu.VMEM((K,), jnp.float32),
                       pltpu.VMEM((K,), jnp.float32),
                       pltpu.VMEM((K, CW), jnp.float32),
                       pltpu.VMEM((16, CW), jnp.float32),
                       pltpu.VMEM((640,), jnp.float32),
                       pltpu.SemaphoreType.DMA],
    )
    return f(srcp, dstp, aep, aq, ak, sh, hwp)


def _fin_body(cout, last, accp_ref, den_ref, b_ref, h_ref):
    num = accp_ref[0, :N, :cout] + accp_ref[1, :N, :cout]
    den = den_ref[...]
    deninv = jnp.where(den > 0.0, 1.0 / den, 0.0)
    h = num * deninv + b_ref[...]
    if not last:
        h = jnp.where(h >= 0.0, h, 0.01 * h)
    h_ref[...] = h


def _fin(accp, dencol, b2d, last):
    cout = b2d.shape[1]
    body = functools.partial(_fin_body, cout, last)
    return pl.pallas_call(
        body,
        in_specs=[pl.BlockSpec((NC, NPAD, CW), lambda: (0, 0, 0)),
                  pl.BlockSpec((N, 1), lambda: (0, 0)),
                  pl.BlockSpec((1, cout), lambda: (0, 0))],
        out_specs=pl.BlockSpec((N, cout), lambda: (0, 0)),
        out_shape=jax.ShapeDtypeStruct((N, cout), jnp.float32),
    )(accp, dencol, b2d)


def _gram_body(h_ref, ht_ref, out_ref):
    out_ref[...] = jnp.dot(h_ref[...], ht_ref[...],
                           preferred_element_type=jnp.float32)


def _gram(h):
    ht = h.T
    return pl.pallas_call(
        _gram_body,
        grid=(10, 10),
        in_specs=[pl.BlockSpec((1024, 128), lambda i, j: (i, 0)),
                  pl.BlockSpec((128, 1024), lambda i, j: (0, j))],
        out_specs=pl.BlockSpec((1024, 1024), lambda i, j: (i, j)),
        out_shape=jax.ShapeDtypeStruct((N, N), jnp.float32),
    )(h, ht)


def kernel(x, edge_index, edge_attr, edge_type, emb, W0, q0, k0, We0, e0, b0,
           W1, q1, k1, We1, e1, b1, W2, q2, k2, We2, e2, b2,
           W3, q3, k3, We3, e3, b3, W4, q4, k4, We4, e4, b4):
    Ws = [W0, W1, W2, W3, W4]
    qs = [q0, q1, q2, q3, q4]
    ks = [k0, k1, k2, k3, k4]
    wts = [We0.T, We1.T, We2.T, We3.T, We4.T]
    es = [e0, e1, e2, e3, e4]
    bs = [b0, b1, b2, b3, b4]

    ae_all, aemax = _stage0(edge_attr, wts, es)

    srcp = jnp.pad(edge_index[0], (0, EP - E))
    dstp = jnp.pad(edge_index[1], (0, EP - E))
    aeps = [jnp.pad(ae_all[:, l], (0, EP - E), constant_values=NEG)
            for l in range(5)]

    idx_pad = jnp.pad(x[:, 0], (0, NPAD - N))
    h = _emb_lookup(emb, idx_pad)[:N]

    for l in range(5):
        am_row = aemax[l:l + 1]
        hwp, aq, ak, sh = _prep(h, Ws[l], qs[l], ks[l], am_row)
        accp, denp = _edge(srcp, dstp, aeps[l],
                           aq.reshape(N), ak.reshape(N), sh.reshape(-1), hwp,
                           Ws[l].shape[1])
        dencol = (denp[0] + denp[1])[:N].reshape(N, 1)
        h = _fin(accp, dencol, bs[l].reshape(1, -1), last=(l == 4))

    return _gram(h)


# speedup vs baseline: 7.8075x; 1.3700x over previous
import functools

import jax
import jax.numpy as jnp
from jax import lax
from jax.experimental import pallas as pl
from jax.experimental.pallas import tpu as pltpu
from jax.experimental.pallas import tpu_sc as plsc

N = 10000
E = 320000
NC, NS = 2, 16
NW = NC * NS
EP = 327680
EW = EP // NW
K = 64
SK = 512
CPS = SK // K
NSUP = EW // SK
NPAD = 10240
NEG = -1.0e30
CW = 128

_MESH = plsc.VectorSubcoreMesh(core_axis_name="c", subcore_axis_name="s",
                               num_cores=NC, num_subcores=NS)
_SC_PARAMS = pltpu.CompilerParams(needs_layout_passes=False)


def _ae_body(ea_ref, wt0, e0, wt1, e1, wt2, e2, wt3, e3, wt4, e4,
             ae_ref, am_ref):
    cols = []
    for wt, ev in ((wt0, e0), (wt1, e1), (wt2, e2), (wt3, e3), (wt4, e4)):
        cols.append(jnp.dot(wt[...], ev[...],
                            preferred_element_type=jnp.float32))
    ce = jnp.concatenate(cols + [cols[0], cols[0], cols[0]], axis=1)
    blk = jnp.dot(ea_ref[...], ce, preferred_element_type=jnp.float32)
    ae_ref[...] = blk

    @pl.when(pl.program_id(0) == 0)
    def _():
        am_ref[...] = jnp.full((8, 128), NEG, jnp.float32)

    for l in range(5):
        ml = jnp.max(blk[:, l])
        prev = am_ref[pl.ds(l, 1), :]
        am_ref[pl.ds(l, 1), :] = jnp.maximum(prev, ml)


def _stage0(edge_attr, wts, es):
    EB = 20000
    args = []
    specs = [pl.BlockSpec((EB, 16), lambda i: (i, 0))]
    for wt, ev in zip(wts, es):
        args += [wt, ev]
        c = wt.shape[1]
        specs += [pl.BlockSpec((16, c), lambda i: (0, 0)),
                  pl.BlockSpec((c, 1), lambda i: (0, 0))]
    return pl.pallas_call(
        _ae_body,
        grid=(E // EB,),
        in_specs=specs,
        out_specs=[pl.BlockSpec((EB, 8), lambda i: (i, 0)),
                   pl.BlockSpec((8, 128), lambda i: (0, 0))],
        out_shape=[jax.ShapeDtypeStruct((E, 8), jnp.float32),
                   jax.ShapeDtypeStruct((8, 128), jnp.float32)],
    )(edge_attr, *args)


def _emb_body(tab_hbm, idx_hbm, out_hbm, idx_v, rows_v, sem):
    wid = lax.axis_index("c") * NS + lax.axis_index("s")

    def step(j, carry):
        base = wid * 320 + j * 80
        pltpu.sync_copy(idx_hbm.at[pl.ds(base, 80)], idx_v)
        pltpu.async_copy(tab_hbm.at[idx_v], rows_v, sem).wait()
        pltpu.sync_copy(rows_v, out_hbm.at[pl.ds(base, 80)])
        return carry

    lax.fori_loop(0, 4, step, 0)


def _emb_lookup(emb, idx_pad):
    f = pl.kernel(
        _emb_body,
        out_type=jax.ShapeDtypeStruct((NPAD, 128), jnp.float32),
        mesh=_MESH,
        compiler_params=_SC_PARAMS,
        scratch_types=[pltpu.VMEM((80,), jnp.int32),
                       pltpu.VMEM((80, 128), jnp.float32),
                       pltpu.SemaphoreType.DMA],
    )
    return f(emb, idx_pad)


def _prep_body(cout, h_ref, w_ref, q_ref, k_ref, am_ref,
               hwp_ref, aq_ref, ak_ref, sh_ref):
    hw = jnp.dot(h_ref[...], w_ref[...], preferred_element_type=jnp.float32)
    hwp_ref[:, :cout] = hw
    if cout < CW:
        hwp_ref[:, cout:] = jnp.zeros((N, CW - cout), jnp.float32)
    aq = jnp.dot(hw, q_ref[...], preferred_element_type=jnp.float32)
    ak = jnp.dot(hw, k_ref[...], preferred_element_type=jnp.float32)
    aq_ref[...] = aq
    ak_ref[...] = ak
    s = jnp.max(aq) + jnp.max(ak) + jnp.max(am_ref[pl.ds(0, 1), :])
    sh_ref[...] = jnp.full((8, 128), s, jnp.float32)


def _prep(h, W, q, k, am_row):
    cin, cout = W.shape
    cp = CW
    body = functools.partial(_prep_body, cout)
    return pl.pallas_call(
        body,
        in_specs=[pl.BlockSpec((N, cin), lambda: (0, 0)),
                  pl.BlockSpec((cin, cout), lambda: (0, 0)),
                  pl.BlockSpec((cout, 1), lambda: (0, 0)),
                  pl.BlockSpec((cout, 1), lambda: (0, 0)),
                  pl.BlockSpec((1, 128), lambda: (0, 0))],
        out_specs=[pl.BlockSpec((N, cp), lambda: (0, 0)),
                   pl.BlockSpec((N, 1), lambda: (0, 0)),
                   pl.BlockSpec((N, 1), lambda: (0, 0)),
                   pl.BlockSpec((8, 128), lambda: (0, 0))],
        out_shape=[jax.ShapeDtypeStruct((N, cp), jnp.float32),
                   jax.ShapeDtypeStruct((N, 1), jnp.float32),
                   jax.ShapeDtypeStruct((N, 1), jnp.float32),
                   jax.ShapeDtypeStruct((8, 128), jnp.float32)],
    )(h, W, q, k, am_row)


def _edge_body(cout, src_hbm, dst_hbm, ae_hbm, aq_hbm, ak_hbm, sh_hbm,
               hwp_hbm, accp_hbm, denp_hbm,
               acc_sh, den_sh, aq_v, ak_v, sh_v, src_s, dst_s, ae_s,
               srcs_v, dsts_v, exs_v, rows0_v, rows1_v, dzb_v,
               sem_g0, sem_g1, sem_s0, sem_s1, sem_d):
    cid = lax.axis_index("c")
    sid = lax.axis_index("s")
    wid = cid * NS + sid
    rows = [rows0_v, rows1_v]
    sem_g = [sem_g0, sem_g1]
    sem_s = [sem_s0, sem_s1]

    zero16 = jnp.zeros((16,), jnp.float32)
    for r in range(16):
        for c in range(CW // 16):
            rows0_v[r, pl.ds(c * 16, 16)] = zero16

    def zclr(j, carry):
        dzb_v[pl.ds(j * 16, 16)] = zero16
        return carry

    lax.fori_loop(0, 40, zclr, 0)
    pltpu.sync_copy(dzb_v, den_sh.at[pl.ds(sid * 640, 640)])

    def zstep(j, carry):
        pltpu.sync_copy(rows0_v.at[pl.ds(0, 16)],
                        acc_sh.at[pl.ds(sid * 640 + j * 16, 16)])
        return carry

    lax.fori_loop(0, 40, zstep, 0)

    pltpu.sync_copy(aq_hbm, aq_v)
    pltpu.sync_copy(ak_hbm, ak_v)
    pltpu.sync_copy(sh_hbm.at[pl.ds(0, 16)], sh_v)
    shift = sh_v[...]
    plsc.subcore_barrier()

    ebase = wid * EW

    def fill_idx(t):
        for g in range(K // 16):
            sl_src = pl.ds(t * K + g * 16, 16)
            sl = pl.ds(g * 16, 16)
            srcs_v[t, sl] = src_s[sl_src]
            dsts_v[t, sl] = dst_s[sl_src]

    def super_chunk(sup, carry):
        off = ebase + sup * SK
        pltpu.sync_copy(src_hbm.at[pl.ds(off, SK)], src_s)
        pltpu.sync_copy(dst_hbm.at[pl.ds(off, SK)], dst_s)
        pltpu.sync_copy(ae_hbm.at[pl.ds(off, SK)], ae_s)

        fill_idx(0)
        gats = [None] * CPS
        scats = [None] * CPS
        dens = [None] * CPS
        gats[0] = pltpu.async_copy(hwp_hbm.at[srcs_v.at[0]], rows[0], sem_g[0])
        for t in range(CPS):
            b = t % 2
            if t + 1 < CPS:
                fill_idx(t + 1)
                if t >= 1:
                    scats[t - 1].wait()
                gats[t + 1] = pltpu.async_copy(
                    hwp_hbm.at[srcs_v.at[t + 1]], rows[(t + 1) % 2],
                    sem_g[(t + 1) % 2])
            for g in range(K // 16):
                sl_src = pl.ds(t * K + g * 16, 16)
                d16 = dst_s[sl_src]
                s16 = src_s[sl_src]
                a = (plsc.load_gather(aq_v, [d16])
                     + plsc.load_gather(ak_v, [s16])
                     + ae_s[sl_src])
                a = jnp.where(a >= 0.0, a, 0.2 * a)
                exs_v[t, pl.ds(g * 16, 16)] = jnp.exp(a - shift)
            gats[t].wait()

            def rgroup(gg, carry2):
                r0 = gg * 8
                for rr in range(8):
                    r = r0 + rr
                    w = plsc.load_gather(
                        exs_v, [jnp.full((16,), t, jnp.int32),
                                jnp.zeros((16,), jnp.int32) + r])
                    for c in range(cout // 16):
                        sl = pl.ds(c * 16, 16)
                        rows[b][r, sl] = rows[b][r, sl] * w
                return carry2

            lax.fori_loop(0, K // 8, rgroup, 0)
            scats[t] = pltpu.async_copy(rows[b], acc_sh.at[dsts_v.at[t]],
                                        sem_s[b], add=True)
            dens[t] = pltpu.async_copy(exs_v.at[t], den_sh.at[dsts_v.at[t]],
                                       sem_d, add=True)
        scats[CPS - 2].wait()
        scats[CPS - 1].wait()
        for t in range(CPS):
            dens[t].wait()
        return carry

    lax.fori_loop(0, NSUP, super_chunk, 0)
    plsc.subcore_barrier()
    pltpu.sync_copy(acc_sh.at[pl.ds(sid * 640, 640)],
                    accp_hbm.at[cid, pl.ds(sid * 640, 640)])
    pltpu.sync_copy(den_sh.at[pl.ds(sid * 640, 640)],
                    denp_hbm.at[cid, pl.ds(sid * 640, 640)])


def _edge(srcp, dstp, aep, aq, ak, sh, hwp, cout):
    body = functools.partial(_edge_body, cout)
    f = pl.kernel(
        body,
        out_type=[jax.ShapeDtypeStruct((NC, NPAD, CW), jnp.float32),
                  jax.ShapeDtypeStruct((NC, NPAD), jnp.float32)],
        mesh=_MESH,
        compiler_params=_SC_PARAMS,
        scratch_types=[pltpu.VMEM_SHARED((NPAD, CW), jnp.float32),
                       pltpu.VMEM_SHARED((NPAD,), jnp.float32),
                       pltpu.VMEM((N,), jnp.float32),
                       pltpu.VMEM((N,), jnp.float32),
                       pltpu.VMEM((16,), jnp.float32),
                       pltpu.VMEM((SK,), jnp.int32),
                       pltpu.VMEM((SK,), jnp.int32),
                       pltpu.VMEM((SK,), jnp.float32),
                       pltpu.VMEM((CPS, K), jnp.int32),
                       pltpu.VMEM((CPS, K), jnp.int32),
                       pltpu.VMEM((CPS, K), jnp.float32),
                       pltpu.VMEM((K, CW), jnp.float32),
                       pltpu.VMEM((K, CW), jnp.float32),
                       pltpu.VMEM((640,), jnp.float32),
                       pltpu.SemaphoreType.DMA,
                       pltpu.SemaphoreType.DMA,
                       pltpu.SemaphoreType.DMA,
                       pltpu.SemaphoreType.DMA,
                       pltpu.SemaphoreType.DMA],
    )
    return f(srcp, dstp, aep, aq, ak, sh, hwp)


def _fin_body(cout, last, accp_ref, den_ref, b_ref, h_ref):
    num = accp_ref[0, :N, :cout] + accp_ref[1, :N, :cout]
    den = den_ref[...]
    deninv = jnp.where(den > 0.0, 1.0 / den, 0.0)
    h = num * deninv + b_ref[...]
    if not last:
        h = jnp.where(h >= 0.0, h, 0.01 * h)
    h_ref[...] = h


def _fin(accp, dencol, b2d, last):
    cout = b2d.shape[1]
    body = functools.partial(_fin_body, cout, last)
    return pl.pallas_call(
        body,
        in_specs=[pl.BlockSpec((NC, NPAD, CW), lambda: (0, 0, 0)),
                  pl.BlockSpec((N, 1), lambda: (0, 0)),
                  pl.BlockSpec((1, cout), lambda: (0, 0))],
        out_specs=pl.BlockSpec((N, cout), lambda: (0, 0)),
        out_shape=jax.ShapeDtypeStruct((N, cout), jnp.float32),
    )(accp, dencol, b2d)


def _gram_body(h_ref, ht_ref, out_ref):
    out_ref[...] = jnp.dot(h_ref[...], ht_ref[...],
                           preferred_element_type=jnp.float32)


def _gram(h):
    ht = h.T
    return pl.pallas_call(
        _gram_body,
        grid=(10, 10),
        in_specs=[pl.BlockSpec((1024, 128), lambda i, j: (i, 0)),
                  pl.BlockSpec((128, 1024), lambda i, j: (0, j))],
        out_specs=pl.BlockSpec((1024, 1024), lambda i, j: (i, j)),
        out_shape=jax.ShapeDtypeStruct((N, N), jnp.float32),
    )(h, ht)


def kernel(x, edge_index, edge_attr, edge_type, emb, W0, q0, k0, We0, e0, b0,
           W1, q1, k1, We1, e1, b1, W2, q2, k2, We2, e2, b2,
           W3, q3, k3, We3, e3, b3, W4, q4, k4, We4, e4, b4):
    Ws = [W0, W1, W2, W3, W4]
    qs = [q0, q1, q2, q3, q4]
    ks = [k0, k1, k2, k3, k4]
    wts = [We0.T, We1.T, We2.T, We3.T, We4.T]
    es = [e0, e1, e2, e3, e4]
    bs = [b0, b1, b2, b3, b4]

    ae_all, aemax = _stage0(edge_attr, wts, es)

    srcp = jnp.pad(edge_index[0], (0, EP - E))
    dstp = jnp.pad(edge_index[1], (0, EP - E))
    aeps = [jnp.pad(ae_all[:, l], (0, EP - E), constant_values=NEG)
            for l in range(5)]

    idx_pad = jnp.pad(x[:, 0], (0, NPAD - N))
    h = _emb_lookup(emb, idx_pad)[:N]

    for l in range(5):
        am_row = aemax[l:l + 1]
        hwp, aq, ak, sh = _prep(h, Ws[l], qs[l], ks[l], am_row)
        accp, denp = _edge(srcp, dstp, aeps[l],
                           aq.reshape(N), ak.reshape(N), sh.reshape(-1), hwp,
                           Ws[l].shape[1])
        dencol = (denp[0] + denp[1])[:N].reshape(N, 1)
        h = _fin(accp, dencol, bs[l].reshape(1, -1), last=(l == 4))

    return _gram(h)


# speedup vs baseline: 7.8425x; 1.0045x over previous
import functools

import jax
import jax.numpy as jnp
from jax import lax
from jax.experimental import pallas as pl
from jax.experimental.pallas import tpu as pltpu
from jax.experimental.pallas import tpu_sc as plsc

N = 10000
E = 320000
NC, NS = 2, 16
NW = NC * NS
EP = 327680
EW = EP // NW
K = 64
SK = 512
CPS = SK // K
NSUP = EW // SK
NPAD = 10240
NEG = -1.0e30
CW = 128

_MESH = plsc.VectorSubcoreMesh(core_axis_name="c", subcore_axis_name="s",
                               num_cores=NC, num_subcores=NS)
_SC_PARAMS = pltpu.CompilerParams(needs_layout_passes=False)


def _ae_body(ea_ref, wt0, e0, wt1, e1, wt2, e2, wt3, e3, wt4, e4,
             ae_ref, am_ref):
    cols = []
    for wt, ev in ((wt0, e0), (wt1, e1), (wt2, e2), (wt3, e3), (wt4, e4)):
        cols.append(jnp.dot(wt[...], ev[...],
                            preferred_element_type=jnp.float32))
    ce = jnp.concatenate(cols + [cols[0], cols[0], cols[0]], axis=1)
    blk = jnp.dot(ea_ref[...], ce, preferred_element_type=jnp.float32)
    ae_ref[...] = blk

    @pl.when(pl.program_id(0) == 0)
    def _():
        am_ref[...] = jnp.full((8, 128), NEG, jnp.float32)

    for l in range(5):
        ml = jnp.max(blk[:, l])
        prev = am_ref[pl.ds(l, 1), :]
        am_ref[pl.ds(l, 1), :] = jnp.maximum(prev, ml)


def _stage0(edge_attr, wts, es):
    EB = 20000
    args = []
    specs = [pl.BlockSpec((EB, 16), lambda i: (i, 0))]
    for wt, ev in zip(wts, es):
        args += [wt, ev]
        c = wt.shape[1]
        specs += [pl.BlockSpec((16, c), lambda i: (0, 0)),
                  pl.BlockSpec((c, 1), lambda i: (0, 0))]
    return pl.pallas_call(
        _ae_body,
        grid=(E // EB,),
        in_specs=specs,
        out_specs=[pl.BlockSpec((EB, 8), lambda i: (i, 0)),
                   pl.BlockSpec((8, 128), lambda i: (0, 0))],
        out_shape=[jax.ShapeDtypeStruct((E, 8), jnp.float32),
                   jax.ShapeDtypeStruct((8, 128), jnp.float32)],
    )(edge_attr, *args)


def _emb_body(tab_hbm, idx_hbm, out_hbm, idx_v, rows_v, sem):
    wid = lax.axis_index("c") * NS + lax.axis_index("s")

    def step(j, carry):
        base = wid * 320 + j * 80
        pltpu.sync_copy(idx_hbm.at[pl.ds(base, 80)], idx_v)
        pltpu.async_copy(tab_hbm.at[idx_v], rows_v, sem).wait()
        pltpu.sync_copy(rows_v, out_hbm.at[pl.ds(base, 80)])
        return carry

    lax.fori_loop(0, 4, step, 0)


def _emb_lookup(emb, idx_pad):
    f = pl.kernel(
        _emb_body,
        out_type=jax.ShapeDtypeStruct((NPAD, 128), jnp.float32),
        mesh=_MESH,
        compiler_params=_SC_PARAMS,
        scratch_types=[pltpu.VMEM((80,), jnp.int32),
                       pltpu.VMEM((80, 128), jnp.float32),
                       pltpu.SemaphoreType.DMA],
    )
    return f(emb, idx_pad)


def _prep_body(cout, h_ref, w_ref, q_ref, k_ref, am_ref,
               hwp_ref, aq_ref, ak_ref, sh_ref):
    hw = jnp.dot(h_ref[...], w_ref[...], preferred_element_type=jnp.float32)
    hwp_ref[:, :cout] = hw
    if cout < CW:
        hwp_ref[:, cout:] = jnp.zeros((N, CW - cout), jnp.float32)
    aq = jnp.dot(hw, q_ref[...], preferred_element_type=jnp.float32)
    ak = jnp.dot(hw, k_ref[...], preferred_element_type=jnp.float32)
    aq_ref[...] = aq
    ak_ref[...] = ak
    s = jnp.max(aq) + jnp.max(ak) + jnp.max(am_ref[pl.ds(0, 1), :])
    sh_ref[...] = jnp.full((8, 128), s, jnp.float32)


def _prep(h, W, q, k, am_row):
    cin, cout = W.shape
    cp = CW
    body = functools.partial(_prep_body, cout)
    return pl.pallas_call(
        body,
        in_specs=[pl.BlockSpec((N, cin), lambda: (0, 0)),
                  pl.BlockSpec((cin, cout), lambda: (0, 0)),
                  pl.BlockSpec((cout, 1), lambda: (0, 0)),
                  pl.BlockSpec((cout, 1), lambda: (0, 0)),
                  pl.BlockSpec((1, 128), lambda: (0, 0))],
        out_specs=[pl.BlockSpec((N, cp), lambda: (0, 0)),
                   pl.BlockSpec((N, 1), lambda: (0, 0)),
                   pl.BlockSpec((N, 1), lambda: (0, 0)),
                   pl.BlockSpec((8, 128), lambda: (0, 0))],
        out_shape=[jax.ShapeDtypeStruct((N, cp), jnp.float32),
                   jax.ShapeDtypeStruct((N, 1), jnp.float32),
                   jax.ShapeDtypeStruct((N, 1), jnp.float32),
                   jax.ShapeDtypeStruct((8, 128), jnp.float32)],
    )(h, W, q, k, am_row)


def _edge_body(cout, src_hbm, dst_hbm, ae_hbm, aq_hbm, ak_hbm, sh_hbm,
               hwp_hbm, accp_hbm, denp_hbm,
               acc_sh, den_sh, aq_v, ak_v, sh_v,
               srcs_v, dsts_v, aes_v, exs_v, rows0_v, rows1_v, dzb_v,
               sem_g0, sem_g1, sem_s0, sem_s1, sem_d):
    cid = lax.axis_index("c")
    sid = lax.axis_index("s")
    wid = cid * NS + sid
    rows = [rows0_v, rows1_v]
    sem_g = [sem_g0, sem_g1]
    sem_s = [sem_s0, sem_s1]

    zero16 = jnp.zeros((16,), jnp.float32)
    for r in range(K):
        for c in range(CW // 16):
            rows0_v[r, pl.ds(c * 16, 16)] = zero16

    def zclr(j, carry):
        dzb_v[pl.ds(j * 16, 16)] = zero16
        return carry

    lax.fori_loop(0, 40, zclr, 0)
    zd = pltpu.async_copy(dzb_v, den_sh.at[pl.ds(sid * 640, 640)], sem_d)
    zs = [pltpu.async_copy(rows0_v,
                           acc_sh.at[pl.ds(sid * 640 + j * K, K)], sem_s0)
          for j in range(640 // K)]
    pltpu.sync_copy(aq_hbm, aq_v)
    pltpu.sync_copy(ak_hbm, ak_v)
    pltpu.sync_copy(sh_hbm.at[pl.ds(0, 16)], sh_v)
    shift = sh_v[...]
    zd.wait()
    for d in zs:
        d.wait()
    plsc.subcore_barrier()

    rbase = wid * (EW // K)

    def super_chunk(sup, carry):
        roff = rbase + sup * CPS
        pltpu.sync_copy(src_hbm.at[pl.ds(roff, CPS)], srcs_v)
        pltpu.sync_copy(dst_hbm.at[pl.ds(roff, CPS)], dsts_v)
        pltpu.sync_copy(ae_hbm.at[pl.ds(roff, CPS)], aes_v)

        gats = [None] * CPS
        scats = [None] * CPS
        dens = [None] * CPS
        gats[0] = pltpu.async_copy(hwp_hbm.at[srcs_v.at[0]], rows[0], sem_g[0])
        for t in range(CPS):
            b = t % 2
            if t + 1 < CPS:
                if t >= 1:
                    scats[t - 1].wait()
                gats[t + 1] = pltpu.async_copy(
                    hwp_hbm.at[srcs_v.at[t + 1]], rows[(t + 1) % 2],
                    sem_g[(t + 1) % 2])
            for g in range(K // 16):
                sl = pl.ds(g * 16, 16)
                d16 = dsts_v[t, sl]
                s16 = srcs_v[t, sl]
                a = (plsc.load_gather(aq_v, [d16])
                     + plsc.load_gather(ak_v, [s16])
                     + aes_v[t, sl])
                a = jnp.where(a >= 0.0, a, 0.2 * a)
                exs_v[t, sl] = jnp.exp(a - shift)
            gats[t].wait()

            def rgroup(gg, carry2):
                r0 = gg * 8
                for rr in range(8):
                    r = r0 + rr
                    w = plsc.load_gather(
                        exs_v, [jnp.full((16,), t, jnp.int32),
                                jnp.zeros((16,), jnp.int32) + r])
                    for c in range(cout // 16):
                        sl = pl.ds(c * 16, 16)
                        rows[b][r, sl] = rows[b][r, sl] * w
                return carry2

            lax.fori_loop(0, K // 8, rgroup, 0)
            scats[t] = pltpu.async_copy(rows[b], acc_sh.at[dsts_v.at[t]],
                                        sem_s[b], add=True)
            dens[t] = pltpu.async_copy(exs_v.at[t], den_sh.at[dsts_v.at[t]],
                                       sem_d, add=True)
        scats[CPS - 2].wait()
        scats[CPS - 1].wait()
        for t in range(CPS):
            dens[t].wait()
        return carry

    lax.fori_loop(0, NSUP, super_chunk, 0)
    plsc.subcore_barrier()
    pltpu.sync_copy(acc_sh.at[pl.ds(sid * 640, 640)],
                    accp_hbm.at[cid, pl.ds(sid * 640, 640)])
    pltpu.sync_copy(den_sh.at[pl.ds(sid * 640, 640)],
                    denp_hbm.at[cid, pl.ds(sid * 640, 640)])


def _edge(src2, dst2, ae2, aq, ak, sh, hwp, cout):
    body = functools.partial(_edge_body, cout)
    f = pl.kernel(
        body,
        out_type=[jax.ShapeDtypeStruct((NC, NPAD, CW), jnp.float32),
                  jax.ShapeDtypeStruct((NC, NPAD), jnp.float32)],
        mesh=_MESH,
        compiler_params=_SC_PARAMS,
        scratch_types=[pltpu.VMEM_SHARED((NPAD, CW), jnp.float32),
                       pltpu.VMEM_SHARED((NPAD,), jnp.float32),
                       pltpu.VMEM((N,), jnp.float32),
                       pltpu.VMEM((N,), jnp.float32),
                       pltpu.VMEM((16,), jnp.float32),
                       pltpu.VMEM((CPS, K), jnp.int32),
                       pltpu.VMEM((CPS, K), jnp.int32),
                       pltpu.VMEM((CPS, K), jnp.float32),
                       pltpu.VMEM((CPS, K), jnp.float32),
                       pltpu.VMEM((K, CW), jnp.float32),
                       pltpu.VMEM((K, CW), jnp.float32),
                       pltpu.VMEM((640,), jnp.float32),
                       pltpu.SemaphoreType.DMA,
                       pltpu.SemaphoreType.DMA,
                       pltpu.SemaphoreType.DMA,
                       pltpu.SemaphoreType.DMA,
                       pltpu.SemaphoreType.DMA],
    )
    return f(src2, dst2, ae2, aq, ak, sh, hwp)


def _fin_body(cout, last, accp_ref, den_ref, b_ref, h_ref):
    num = accp_ref[0, :N, :cout] + accp_ref[1, :N, :cout]
    den = den_ref[...]
    deninv = jnp.where(den > 0.0, 1.0 / den, 0.0)
    h = num * deninv + b_ref[...]
    if not last:
        h = jnp.where(h >= 0.0, h, 0.01 * h)
    h_ref[...] = h


def _fin(accp, dencol, b2d, last):
    cout = b2d.shape[1]
    body = functools.partial(_fin_body, cout, last)
    return pl.pallas_call(
        body,
        in_specs=[pl.BlockSpec((NC, NPAD, CW), lambda: (0, 0, 0)),
                  pl.BlockSpec((N, 1), lambda: (0, 0)),
                  pl.BlockSpec((1, cout), lambda: (0, 0))],
        out_specs=pl.BlockSpec((N, cout), lambda: (0, 0)),
        out_shape=jax.ShapeDtypeStruct((N, cout), jnp.float32),
    )(accp, dencol, b2d)


def _gram_body(h_ref, ht_ref, out_ref):
    out_ref[...] = jnp.dot(h_ref[...], ht_ref[...],
                           preferred_element_type=jnp.float32)


def _gram(h):
    ht = h.T
    return pl.pallas_call(
        _gram_body,
        grid=(10, 10),
        in_specs=[pl.BlockSpec((1024, 128), lambda i, j: (i, 0)),
                  pl.BlockSpec((128, 1024), lambda i, j: (0, j))],
        out_specs=pl.BlockSpec((1024, 1024), lambda i, j: (i, j)),
        out_shape=jax.ShapeDtypeStruct((N, N), jnp.float32),
    )(h, ht)


def kernel(x, edge_index, edge_attr, edge_type, emb, W0, q0, k0, We0, e0, b0,
           W1, q1, k1, We1, e1, b1, W2, q2, k2, We2, e2, b2,
           W3, q3, k3, We3, e3, b3, W4, q4, k4, We4, e4, b4):
    Ws = [W0, W1, W2, W3, W4]
    qs = [q0, q1, q2, q3, q4]
    ks = [k0, k1, k2, k3, k4]
    wts = [We0.T, We1.T, We2.T, We3.T, We4.T]
    es = [e0, e1, e2, e3, e4]
    bs = [b0, b1, b2, b3, b4]

    ae_all, aemax = _stage0(edge_attr, wts, es)

    srcp = jnp.pad(edge_index[0], (0, EP - E)).reshape(EP // K, K)
    dstp = jnp.pad(edge_index[1], (0, EP - E)).reshape(EP // K, K)
    aeps = [jnp.pad(ae_all[:, l], (0, EP - E),
                    constant_values=NEG).reshape(EP // K, K)
            for l in range(5)]

    idx_pad = jnp.pad(x[:, 0], (0, NPAD - N))
    h = _emb_lookup(emb, idx_pad)[:N]

    for l in range(5):
        am_row = aemax[l:l + 1]
        hwp, aq, ak, sh = _prep(h, Ws[l], qs[l], ks[l], am_row)
        accp, denp = _edge(srcp, dstp, aeps[l],
                           aq.reshape(N), ak.reshape(N), sh.reshape(-1), hwp,
                           Ws[l].shape[1])
        dencol = (denp[0] + denp[1])[:N].reshape(N, 1)
        h = _fin(accp, dencol, bs[l].reshape(1, -1), last=(l == 4))

    return _gram(h)


# speedup vs baseline: 8.4890x; 1.0824x over previous
import functools

import jax
import jax.numpy as jnp
from jax import lax
from jax.experimental import pallas as pl
from jax.experimental.pallas import tpu as pltpu
from jax.experimental.pallas import tpu_sc as plsc

N = 10000
E = 320000
NC, NS = 2, 16
NW = NC * NS
EP = 327680
EW = EP // NW
K = 64
SK = 512
CPS = SK // K
NSUP = EW // SK
NPAD = 10240
NEG = -1.0e30
CW = 128

_MESH = plsc.VectorSubcoreMesh(core_axis_name="c", subcore_axis_name="s",
                               num_cores=NC, num_subcores=NS)
_SC_PARAMS = pltpu.CompilerParams(needs_layout_passes=False)


def _ae_body(ea_ref, wt0, e0, wt1, e1, wt2, e2, wt3, e3, wt4, e4,
             ae_ref, am_ref):
    cols = []
    for wt, ev in ((wt0, e0), (wt1, e1), (wt2, e2), (wt3, e3), (wt4, e4)):
        cols.append(jnp.dot(wt[...], ev[...],
                            preferred_element_type=jnp.float32))
    ce = jnp.concatenate(cols + [cols[0], cols[0], cols[0]], axis=1)
    blk = jnp.dot(ea_ref[...], ce, preferred_element_type=jnp.float32)
    ae_ref[...] = blk

    @pl.when(pl.program_id(0) == 0)
    def _():
        am_ref[...] = jnp.full((8, 128), NEG, jnp.float32)

    for l in range(5):
        ml = jnp.max(blk[:, l])
        prev = am_ref[pl.ds(l, 1), :]
        am_ref[pl.ds(l, 1), :] = jnp.maximum(prev, ml)


def _stage0(edge_attr, wts, es):
    EB = 20000
    args = []
    specs = [pl.BlockSpec((EB, 16), lambda i: (i, 0))]
    for wt, ev in zip(wts, es):
        args += [wt, ev]
        c = wt.shape[1]
        specs += [pl.BlockSpec((16, c), lambda i: (0, 0)),
                  pl.BlockSpec((c, 1), lambda i: (0, 0))]
    return pl.pallas_call(
        _ae_body,
        grid=(E // EB,),
        in_specs=specs,
        out_specs=[pl.BlockSpec((EB, 8), lambda i: (i, 0)),
                   pl.BlockSpec((8, 128), lambda i: (0, 0))],
        out_shape=[jax.ShapeDtypeStruct((E, 8), jnp.float32),
                   jax.ShapeDtypeStruct((8, 128), jnp.float32)],
    )(edge_attr, *args)


def _emb_body(tab_hbm, idx_hbm, out_hbm, idx_v, rows_v, sem):
    wid = lax.axis_index("c") * NS + lax.axis_index("s")

    def step(j, carry):
        base = wid * 320 + j * 80
        pltpu.sync_copy(idx_hbm.at[pl.ds(base, 80)], idx_v)
        pltpu.async_copy(tab_hbm.at[idx_v], rows_v, sem).wait()
        pltpu.sync_copy(rows_v, out_hbm.at[pl.ds(base, 80)])
        return carry

    lax.fori_loop(0, 4, step, 0)


def _emb_lookup(emb, idx_pad):
    f = pl.kernel(
        _emb_body,
        out_type=jax.ShapeDtypeStruct((NPAD, 128), jnp.float32),
        mesh=_MESH,
        compiler_params=_SC_PARAMS,
        scratch_types=[pltpu.VMEM((80,), jnp.int32),
                       pltpu.VMEM((80, 128), jnp.float32),
                       pltpu.SemaphoreType.DMA],
    )
    return f(emb, idx_pad)


def _prep_body(cout, h_ref, w_ref, q_ref, k_ref, am_ref,
               hwp_ref, aq_ref, ak_ref, sh_ref):
    hw = jnp.dot(h_ref[...], w_ref[...], preferred_element_type=jnp.float32)
    hwp_ref[:, :cout] = hw
    if cout < CW:
        hwp_ref[:, cout:] = jnp.zeros((N, CW - cout), jnp.float32)
    aq = jnp.dot(hw, q_ref[...], preferred_element_type=jnp.float32)
    ak = jnp.dot(hw, k_ref[...], preferred_element_type=jnp.float32)
    aq_ref[...] = aq
    ak_ref[...] = ak
    s = jnp.max(aq) + jnp.max(ak) + jnp.max(am_ref[pl.ds(0, 1), :])
    sh_ref[...] = jnp.full((8, 128), s, jnp.float32)


def _prep(h, W, q, k, am_row):
    cin, cout = W.shape
    cp = CW
    body = functools.partial(_prep_body, cout)
    return pl.pallas_call(
        body,
        in_specs=[pl.BlockSpec((N, cin), lambda: (0, 0)),
                  pl.BlockSpec((cin, cout), lambda: (0, 0)),
                  pl.BlockSpec((cout, 1), lambda: (0, 0)),
                  pl.BlockSpec((cout, 1), lambda: (0, 0)),
                  pl.BlockSpec((1, 128), lambda: (0, 0))],
        out_specs=[pl.BlockSpec((N, cp), lambda: (0, 0)),
                   pl.BlockSpec((N, 1), lambda: (0, 0)),
                   pl.BlockSpec((N, 1), lambda: (0, 0)),
                   pl.BlockSpec((8, 128), lambda: (0, 0))],
        out_shape=[jax.ShapeDtypeStruct((N, cp), jnp.float32),
                   jax.ShapeDtypeStruct((N, 1), jnp.float32),
                   jax.ShapeDtypeStruct((N, 1), jnp.float32),
                   jax.ShapeDtypeStruct((8, 128), jnp.float32)],
    )(h, W, q, k, am_row)


def _edge_body(cout, src_hbm, dst_hbm, ae_hbm, aq_hbm, ak_hbm, sh_hbm,
               hwp_hbm, accp_hbm, denp_hbm,
               acc_sh, den_sh, aq_v, ak_v, sh_v,
               srcs_v, dsts_v, aes_v, exs_v, rows0_v, rows1_v, dzb_v,
               sem_g0, sem_g1, sem_s0, sem_s1, sem_d):
    cid = lax.axis_index("c")
    sid = lax.axis_index("s")
    wid = cid * NS + sid
    rows = [rows0_v, rows1_v]
    sem_g = [sem_g0, sem_g1]
    sem_s = [sem_s0, sem_s1]

    zero16 = jnp.zeros((16,), jnp.float32)
    for r in range(K):
        for c in range(CW // 16):
            rows0_v[r, pl.ds(c * 16, 16)] = zero16

    def zclr(j, carry):
        dzb_v[pl.ds(j * 16, 16)] = zero16
        return carry

    lax.fori_loop(0, 40, zclr, 0)
    zd = pltpu.async_copy(dzb_v, den_sh.at[pl.ds(sid * 640, 640)], sem_d)
    zs = [pltpu.async_copy(rows0_v,
                           acc_sh.at[pl.ds(sid * 640 + j * K, K)], sem_s0)
          for j in range(640 // K)]
    pltpu.sync_copy(aq_hbm, aq_v)
    pltpu.sync_copy(ak_hbm, ak_v)
    pltpu.sync_copy(sh_hbm.at[pl.ds(0, 16)], sh_v)
    shift = sh_v[...]
    zd.wait()
    for d in zs:
        d.wait()
    plsc.subcore_barrier()

    rbase = wid * (EW // K)

    def super_chunk(sup, carry):
        roff = rbase + sup * CPS
        pltpu.sync_copy(src_hbm.at[pl.ds(roff, CPS)], srcs_v)
        pltpu.sync_copy(dst_hbm.at[pl.ds(roff, CPS)], dsts_v)
        pltpu.sync_copy(ae_hbm.at[pl.ds(roff, CPS)], aes_v)

        gats = [None] * CPS
        scats = [None] * CPS
        dens = [None] * CPS
        gats[0] = pltpu.async_copy(hwp_hbm.at[srcs_v.at[0]], rows[0], sem_g[0])
        for t in range(CPS):
            b = t % 2
            if t + 1 < CPS:
                if t >= 1:
                    scats[t - 1].wait()
                gats[t + 1] = pltpu.async_copy(
                    hwp_hbm.at[srcs_v.at[t + 1]], rows[(t + 1) % 2],
                    sem_g[(t + 1) % 2])
            for g in range(K // 16):
                sl = pl.ds(g * 16, 16)
                d16 = dsts_v[t, sl]
                s16 = srcs_v[t, sl]
                a = (plsc.load_gather(aq_v, [d16])
                     + plsc.load_gather(ak_v, [s16])
                     + aes_v[t, sl])
                a = jnp.where(a >= 0.0, a, 0.2 * a)
                exs_v[t, sl] = jnp.exp(a - shift)
            gats[t].wait()

            def rgroup(gg, carry2):
                r0 = gg * 8
                for rr in range(8):
                    r = r0 + rr
                    w = plsc.load_gather(
                        exs_v, [jnp.full((16,), t, jnp.int32),
                                jnp.zeros((16,), jnp.int32) + r])
                    for c in range(cout // 16):
                        sl = pl.ds(c * 16, 16)
                        rows[b][r, sl] = rows[b][r, sl] * w
                return carry2

            lax.fori_loop(0, K // 8, rgroup, 0)
            scats[t] = pltpu.async_copy(rows[b], acc_sh.at[dsts_v.at[t]],
                                        sem_s[b], add=True)
            dens[t] = pltpu.async_copy(exs_v.at[t], den_sh.at[dsts_v.at[t]],
                                       sem_d, add=True)
        scats[CPS - 2].wait()
        scats[CPS - 1].wait()
        for t in range(CPS):
            dens[t].wait()
        return carry

    lax.fori_loop(0, NSUP, super_chunk, 0)
    plsc.subcore_barrier()
    pltpu.sync_copy(acc_sh.at[pl.ds(sid * 640, 640)],
                    accp_hbm.at[cid, pl.ds(sid * 640, 640)])
    pltpu.sync_copy(den_sh.at[pl.ds(sid * 640, 640)],
                    denp_hbm.at[cid, pl.ds(sid * 640, 640)])


def _edge(src2, dst2, ae2, aq, ak, sh, hwp, cout):
    body = functools.partial(_edge_body, cout)
    f = pl.kernel(
        body,
        out_type=[jax.ShapeDtypeStruct((NC, NPAD, CW), jnp.float32),
                  jax.ShapeDtypeStruct((NC, NPAD), jnp.float32)],
        mesh=_MESH,
        compiler_params=_SC_PARAMS,
        scratch_types=[pltpu.VMEM_SHARED((NPAD, CW), jnp.float32),
                       pltpu.VMEM_SHARED((NPAD,), jnp.float32),
                       pltpu.VMEM((N,), jnp.float32),
                       pltpu.VMEM((N,), jnp.float32),
                       pltpu.VMEM((16,), jnp.float32),
                       pltpu.VMEM((CPS, K), jnp.int32),
                       pltpu.VMEM((CPS, K), jnp.int32),
                       pltpu.VMEM((CPS, K), jnp.float32),
                       pltpu.VMEM((CPS, K), jnp.float32),
                       pltpu.VMEM((K, CW), jnp.float32),
                       pltpu.VMEM((K, CW), jnp.float32),
                       pltpu.VMEM((640,), jnp.float32),
                       pltpu.SemaphoreType.DMA,
                       pltpu.SemaphoreType.DMA,
                       pltpu.SemaphoreType.DMA,
                       pltpu.SemaphoreType.DMA,
                       pltpu.SemaphoreType.DMA],
    )
    return f(src2, dst2, ae2, aq, ak, sh, hwp)


def _fin_body(cout, last, accp_ref, den_ref, b_ref, h_ref):
    num = accp_ref[0, :N, :cout] + accp_ref[1, :N, :cout]
    den = den_ref[...]
    deninv = jnp.where(den > 0.0, 1.0 / den, 0.0)
    h = num * deninv + b_ref[...]
    if not last:
        h = jnp.where(h >= 0.0, h, 0.01 * h)
    h_ref[...] = h


def _finprep_body(cout, cout2, accp_ref, den_ref, b_ref, w_ref, q_ref, k_ref,
                  am_ref, hwp_ref, aq_ref, ak_ref, sh_ref):
    num = accp_ref[0, :N, :cout] + accp_ref[1, :N, :cout]
    den = den_ref[...]
    deninv = jnp.where(den > 0.0, 1.0 / den, 0.0)
    h = num * deninv + b_ref[...]
    h = jnp.where(h >= 0.0, h, 0.01 * h)
    hw = jnp.dot(h, w_ref[...], preferred_element_type=jnp.float32)
    hwp_ref[:, :cout2] = hw
    if cout2 < CW:
        hwp_ref[:, cout2:] = jnp.zeros((N, CW - cout2), jnp.float32)
    aq = jnp.dot(hw, q_ref[...], preferred_element_type=jnp.float32)
    ak = jnp.dot(hw, k_ref[...], preferred_element_type=jnp.float32)
    aq_ref[...] = aq
    ak_ref[...] = ak
    s = jnp.max(aq) + jnp.max(ak) + jnp.max(am_ref[pl.ds(0, 1), :])
    sh_ref[...] = jnp.full((8, 128), s, jnp.float32)


def _finprep(accp, dencol, b2d, W, q, k, am_row):
    cout = b2d.shape[1]
    cin, cout2 = W.shape
    body = functools.partial(_finprep_body, cout, cout2)
    return pl.pallas_call(
        body,
        in_specs=[pl.BlockSpec((NC, NPAD, CW), lambda: (0, 0, 0)),
                  pl.BlockSpec((N, 1), lambda: (0, 0)),
                  pl.BlockSpec((1, cout), lambda: (0, 0)),
                  pl.BlockSpec((cin, cout2), lambda: (0, 0)),
                  pl.BlockSpec((cout2, 1), lambda: (0, 0)),
                  pl.BlockSpec((cout2, 1), lambda: (0, 0)),
                  pl.BlockSpec((1, 128), lambda: (0, 0))],
        out_specs=[pl.BlockSpec((N, CW), lambda: (0, 0)),
                   pl.BlockSpec((N, 1), lambda: (0, 0)),
                   pl.BlockSpec((N, 1), lambda: (0, 0)),
                   pl.BlockSpec((8, 128), lambda: (0, 0))],
        out_shape=[jax.ShapeDtypeStruct((N, CW), jnp.float32),
                   jax.ShapeDtypeStruct((N, 1), jnp.float32),
                   jax.ShapeDtypeStruct((N, 1), jnp.float32),
                   jax.ShapeDtypeStruct((8, 128), jnp.float32)],
    )(accp, dencol, b2d, W, q, k, am_row)


def _fin(accp, dencol, b2d, last):
    cout = b2d.shape[1]
    body = functools.partial(_fin_body, cout, last)
    return pl.pallas_call(
        body,
        in_specs=[pl.BlockSpec((NC, NPAD, CW), lambda: (0, 0, 0)),
                  pl.BlockSpec((N, 1), lambda: (0, 0)),
                  pl.BlockSpec((1, cout), lambda: (0, 0))],
        out_specs=pl.BlockSpec((N, cout), lambda: (0, 0)),
        out_shape=jax.ShapeDtypeStruct((N, cout), jnp.float32),
    )(accp, dencol, b2d)


def _gram_body(h_ref, ht_ref, out_ref):
    out_ref[...] = jnp.dot(h_ref[...], ht_ref[...],
                           preferred_element_type=jnp.float32)


def _gram(h):
    ht = h.T
    return pl.pallas_call(
        _gram_body,
        grid=(10, 10),
        in_specs=[pl.BlockSpec((1024, 128), lambda i, j: (i, 0)),
                  pl.BlockSpec((128, 1024), lambda i, j: (0, j))],
        out_specs=pl.BlockSpec((1024, 1024), lambda i, j: (i, j)),
        out_shape=jax.ShapeDtypeStruct((N, N), jnp.float32),
    )(h, ht)


def kernel(x, edge_index, edge_attr, edge_type, emb, W0, q0, k0, We0, e0, b0,
           W1, q1, k1, We1, e1, b1, W2, q2, k2, We2, e2, b2,
           W3, q3, k3, We3, e3, b3, W4, q4, k4, We4, e4, b4):
    Ws = [W0, W1, W2, W3, W4]
    qs = [q0, q1, q2, q3, q4]
    ks = [k0, k1, k2, k3, k4]
    wts = [We0.T, We1.T, We2.T, We3.T, We4.T]
    es = [e0, e1, e2, e3, e4]
    bs = [b0, b1, b2, b3, b4]

    ae_all, aemax = _stage0(edge_attr, wts, es)

    srcp = jnp.pad(edge_index[0], (0, EP - E)).reshape(EP // K, K)
    dstp = jnp.pad(edge_index[1], (0, EP - E)).reshape(EP // K, K)
    aeps = [jnp.pad(ae_all[:, l], (0, EP - E),
                    constant_values=NEG).reshape(EP // K, K)
            for l in range(5)]

    idx_pad = jnp.pad(x[:, 0], (0, NPAD - N))
    h = _emb_lookup(emb, idx_pad)[:N]

    hwp, aq, ak, sh = _prep(h, Ws[0], qs[0], ks[0], aemax[0:1])
    for l in range(5):
        accp, denp = _edge(srcp, dstp, aeps[l],
                           aq.reshape(N), ak.reshape(N), sh.reshape(-1), hwp,
                           Ws[l].shape[1])
        dencol = (denp[0] + denp[1])[:N].reshape(N, 1)
        if l < 4:
            hwp, aq, ak, sh = _finprep(accp, dencol, bs[l].reshape(1, -1),
                                       Ws[l + 1], qs[l + 1], ks[l + 1],
                                       aemax[l + 1:l + 2])
        else:
            h = _fin(accp, dencol, bs[l].reshape(1, -1), last=True)

    return _gram(h)
